# baseline (speedup 1.0000x reference)
import functools

import jax
import jax.numpy as jnp
from jax import lax
from jax.experimental import pallas as pl
from jax.experimental.pallas import tpu as pltpu
from jax.experimental.pallas import tpu_sc as plsc

B = 16384
D = 64
PR = 2 * D
LANES = 16
NC, NS = 2, 16
NW = NC * NS
ROWS = B // NW
CHUNK = 64
NCHUNK = ROWS // CHUNK
MARGIN = 1.0
K = D // LANES


def _rsqrt(x):
    i = plsc.bitcast(x, jnp.int32)
    i = jnp.int32(0x5F3759DF) - lax.shift_right_logical(i, 1)
    y = plsc.bitcast(i, jnp.float32)
    for _ in range(3):
        y = y * (1.5 - 0.5 * x * y * y)
    return y


def _load4(buf, r, base):
    out = []
    for w in range(2):
        xi = buf[r, pl.ds(base + 16 * w, 16)]
        out.append(plsc.bitcast(lax.shift_left(xi, 16), jnp.float32))
        out.append(plsc.bitcast(xi & jnp.int32(-65536), jnp.float32))
    return out


def _transfer_row(buf, rtk, r, p):
    ek = _load4(buf, r, p)
    tk = _load4(buf, r, p + 32)
    d = ek[0] * tk[0]
    for k in range(1, K):
        d = d + ek[k] * tk[k]
    dsum = jnp.full((LANES,), jnp.sum(d), jnp.float32)
    vk = [ek[k] + dsum * rtk[k] for k in range(K)]
    s2 = vk[0] * vk[0]
    for k in range(1, K):
        s2 = s2 + vk[k] * vk[k]
    s2s = jnp.full((LANES,), jnp.sum(s2), jnp.float32)
    y = _rsqrt(s2s)
    norm = s2s * y
    inv = 1.0 / jnp.maximum(norm, 1e-12)
    return [vk[k] * inv for k in range(K)]


def _side_score(hb, tb, rb, r, ph_, pt_, pr_):
    rtk = _load4(rb, r, pr_ + 32)
    hk = _transfer_row(hb, rtk, r, ph_)
    tk = _transfer_row(tb, rtk, r, pt_)
    rek = _load4(rb, r, pr_)
    acc = None
    for k in range(K):
        term = jnp.abs(hk[k] + rek[k] - tk[k])
        acc = term if acc is None else acc + term
    return jnp.sum(acc)


def _disc_body(iph, ipt, ipr, inh, intt, inr,
               aph, apt, apr, anh, antt, anr, takef,
               entc, relc,
               nscore_out, partial_out,
               iv0, iv1, iv2, iv3, iv4, iv5,
               pv0, pv1, pv2, pv3, pv4, pv5,
               b0, b1, b2, b3, b4, b5,
               c0, c1, c2, c3, c4, c5,
               take_v, ns_buf, loss_buf, isem, gsem0, gsem1):
    idx_v = [iv0, iv1, iv2, iv3, iv4, iv5]
    par_v = [pv0, pv1, pv2, pv3, pv4, pv5]
    bufs = [[b0, b1, b2, b3, b4, b5], [c0, c1, c2, c3, c4, c5]]
    gsems = [gsem0, gsem1]
    tabs = [entc, entc, relc, entc, entc, relc]
    wid = lax.axis_index("s") * NC + lax.axis_index("c")
    base = wid * ROWS
    lane = lax.iota(jnp.int32, LANES)

    pre_cps = [
        pltpu.async_copy(src.at[pl.ds(base, ROWS)], dst, isem)
        for src, dst in ((iph, iv0), (ipt, iv1), (ipr, iv2),
                         (inh, iv3), (intt, iv4), (inr, iv5),
                         (aph, pv0), (apt, pv1), (apr, pv2),
                         (anh, pv3), (antt, pv4), (anr, pv5),
                         (takef, take_v))
    ]
    for cp in pre_cps:
        cp.wait()

    def gather_descs(c, s):
        sl = pl.ds(c * CHUNK, CHUNK)
        return [
            pltpu.make_async_copy(tabs[k].at[idx_v[k].at[sl]],
                                  bufs[s][k], gsems[s])
            for k in range(6)
        ]

    def issue(c, s):
        for d in gather_descs(c, s):
            d.start()

    def drain(c, s):
        for d in gather_descs(c, s):
            d.wait()

    def compute(c, s, lossv_c):
        phb, ptb, prb, nhb, ntb, nrb = bufs[s]

        def group_body(g, lv):
            row0 = c * CHUNK + g * LANES
            pv = [par_v[j][pl.ds(row0, LANES)] for j in range(6)]
            zi = jnp.zeros((LANES,), jnp.int32)

            def row_body(i, carry):
                nsv, psv = carry
                r = g * LANES + i
                onehot = lane == i
                pars = [jnp.sum(jnp.where(onehot, pv[j], zi))
                        for j in range(6)]
                p_s = _side_score(phb, ptb, prb, r,
                                  pars[0], pars[1], pars[2])
                n_s = _side_score(nhb, ntb, nrb, r,
                                  pars[3], pars[4], pars[5])
                nsv = jnp.where(onehot,
                                jnp.full((LANES,), -n_s, jnp.float32), nsv)
                psv = jnp.where(onehot,
                                jnp.full((LANES,), p_s, jnp.float32), psv)
                return nsv, psv

            z = jnp.zeros((LANES,), jnp.float32)
            nsv, psv = lax.fori_loop(0, LANES, row_body, (z, z))
            tkv = take_v[pl.ds(row0, LANES)]
            lv = lv + jnp.maximum(0.0, psv + nsv + MARGIN) * tkv
            ns_buf[pl.ds(row0, LANES)] = nsv
            return lv

        return lax.fori_loop(0, CHUNK // LANES, group_body, lossv_c)

    issue(0, 0)

    def pair_body(p, lossv_c):
        c0 = p * 2
        drain(c0, 0)
        issue(c0 + 1, 1)
        lossv_c = compute(c0, 0, lossv_c)
        drain(c0 + 1, 1)

        @pl.when(p < NCHUNK // 2 - 1)
        def _():
            issue(c0 + 2, 0)

        return compute(c0 + 1, 1, lossv_c)

    lossv = lax.fori_loop(0, NCHUNK // 2, pair_body,
                          jnp.zeros((LANES,), jnp.float32))

    pltpu.sync_copy(ns_buf, nscore_out.at[pl.ds(base, ROWS)])
    loss_buf[...] = lossv
    pltpu.sync_copy(loss_buf, partial_out.at[wid])


_disc = functools.partial(
    pl.kernel,
    mesh=plsc.VectorSubcoreMesh(core_axis_name="c", subcore_axis_name="s"),
    compiler_params=pltpu.CompilerParams(needs_layout_passes=False),
    out_type=[
        jax.ShapeDtypeStruct((B,), jnp.float32),
        jax.ShapeDtypeStruct((NW, LANES), jnp.float32),
    ],
    scratch_types=(
        [pltpu.VMEM((ROWS,), jnp.int32) for _ in range(12)]
        + [pltpu.VMEM((CHUNK, PR), jnp.int32) for _ in range(12)]
        + [pltpu.VMEM((ROWS,), jnp.float32),
           pltpu.VMEM((ROWS,), jnp.float32),
           pltpu.VMEM((LANES,), jnp.float32),
           pltpu.SemaphoreType.DMA,
           pltpu.SemaphoreType.DMA,
           pltpu.SemaphoreType.DMA]
    ),
)(_disc_body)


def _bf16_words(x):
    bits = lax.bitcast_convert_type(x, jnp.int32)
    rb = lax.shift_right_logical(
        bits + 0x7FFF + (lax.shift_right_logical(bits, 16) & 1), 16)
    return rb[:, 0:D // 2] | lax.shift_left(rb[:, D // 2:D], 16)


def _repack2_body(xe_ref, xt_ref, o_ref, *, half):
    we = _bf16_words(xe_ref[...].T)
    wt = _bf16_words(xt_ref[...].T)
    o_ref[:, 0:32] = we[0:half]
    o_ref[:, 32:64] = wt[0:half]
    o_ref[:, 64:96] = we[half:2 * half]
    o_ref[:, 96:128] = wt[half:2 * half]


def _make_repack2(n_rows, blk):
    half = blk // 2
    grid = -(-n_rows // blk)
    return pl.pallas_call(
        functools.partial(_repack2_body, half=half),
        grid=(grid,),
        in_specs=[pl.BlockSpec((D, blk), lambda i: (0, i)),
                  pl.BlockSpec((D, blk), lambda i: (0, i))],
        out_specs=pl.BlockSpec((half, PR), lambda i: (i, 0)),
        out_shape=jax.ShapeDtypeStruct((grid * half, PR), jnp.int32),
    )


_ENT_BLK = 16384
_REL_BLK = 1000
_repack_ent = _make_repack2(1000000, _ENT_BLK)
_repack_rel = _make_repack2(1000, _REL_BLK)


def _sum_body(p_ref, o_ref):
    o_ref[0, 0] = jnp.sum(p_ref[...])


_sum_partials = pl.pallas_call(
    _sum_body,
    out_shape=jax.ShapeDtypeStruct((1, 1), jnp.float32),
    out_specs=pl.BlockSpec(memory_space=pltpu.SMEM),
)


def kernel(pos_h, pos_r, pos_t, neg_h, neg_r, neg_t, take,
           ent_emb_w, rel_emb_w, ent_transfer_w, rel_transfer_w):
    def map_idx(s, blk):
        s = s.astype(jnp.int32)
        half = blk // 2
        u = s % blk
        return ((s // blk) * half + u % half,
                (u >= half).astype(jnp.int32) * D)

    blks = (_ENT_BLK, _ENT_BLK, _REL_BLK, _ENT_BLK, _ENT_BLK, _REL_BLK)
    mapped = [map_idx(s, blk) for s, blk in
              zip((pos_h, pos_t, pos_r, neg_h, neg_t, neg_r), blks)]
    phys = [m[0] for m in mapped]
    pars = [m[1] for m in mapped]
    takef = take.astype(jnp.float32)
    entc = _repack_ent(ent_emb_w.T, ent_transfer_w.T)
    relc = _repack_rel(rel_emb_w.T, rel_transfer_w.T)
    nscore, partials = _disc(*phys, *pars, takef, entc, relc)
    loss = _sum_partials(partials)[0, 0]
    return (loss, nscore)


# speedup vs baseline: 1.4754x; 1.4754x over previous
import functools

import jax
import jax.numpy as jnp
from jax import lax
from jax.experimental import pallas as pl
from jax.experimental.pallas import tpu as pltpu
from jax.experimental.pallas import tpu_sc as plsc

B = 16384
D = 64
PR = 2 * D
LANES = 16
NC, NS = 2, 16
NW = NC * NS
ROWS = B // NW
CHUNK = 64
NCHUNK = ROWS // CHUNK
MARGIN = 1.0
K = D // LANES


def _rsqrt(x):
    i = plsc.bitcast(x, jnp.int32)
    i = jnp.int32(0x5F3759DF) - lax.shift_right_logical(i, 1)
    y = plsc.bitcast(i, jnp.float32)
    for _ in range(3):
        y = y * (1.5 - 0.5 * x * y * y)
    return y


def _transfer_row(buf, rtk, r):
    ek = [buf[r, pl.ds(16 * k, 16)] for k in range(K)]
    tk = [buf[r, pl.ds(D + 16 * k, 16)] for k in range(K)]
    d = ek[0] * tk[0]
    for k in range(1, K):
        d = d + ek[k] * tk[k]
    dsum = jnp.full((LANES,), jnp.sum(d), jnp.float32)
    vk = [ek[k] + dsum * rtk[k] for k in range(K)]
    s2 = vk[0] * vk[0]
    for k in range(1, K):
        s2 = s2 + vk[k] * vk[k]
    s2s = jnp.full((LANES,), jnp.sum(s2), jnp.float32)
    y = _rsqrt(s2s)
    norm = s2s * y
    inv = 1.0 / jnp.maximum(norm, 1e-12)
    return [vk[k] * inv for k in range(K)]


def _side_score(hb, tb, rb, r):
    rtk = [rb[r, pl.ds(D + 16 * k, 16)] for k in range(K)]
    hk = _transfer_row(hb, rtk, r)
    tk = _transfer_row(tb, rtk, r)
    acc = None
    for k in range(K):
        rek = rb[r, pl.ds(16 * k, 16)]
        term = jnp.abs(hk[k] + rek - tk[k])
        acc = term if acc is None else acc + term
    return jnp.sum(acc)


def _disc_body(iph, ipt, ipr, inh, intt, inr, takef,
               entc, relc,
               nscore_out, partial_out,
               iv0, iv1, iv2, iv3, iv4, iv5,
               b0, b1, b2, b3, b4, b5,
               c0, c1, c2, c3, c4, c5,
               take_v, ns_buf, loss_buf, isem, gsem0, gsem1):
    idx_v = [iv0, iv1, iv2, iv3, iv4, iv5]
    bufs = [[b0, b1, b2, b3, b4, b5], [c0, c1, c2, c3, c4, c5]]
    gsems = [gsem0, gsem1]
    tabs = [entc, entc, relc, entc, entc, relc]
    wid = lax.axis_index("s") * NC + lax.axis_index("c")
    base = wid * ROWS
    lane = lax.iota(jnp.int32, LANES)

    pre_cps = [
        pltpu.async_copy(src.at[pl.ds(base, ROWS)], dst, isem)
        for src, dst in ((iph, iv0), (ipt, iv1), (ipr, iv2),
                         (inh, iv3), (intt, iv4), (inr, iv5),
                         (takef, take_v))
    ]
    for cp in pre_cps:
        cp.wait()

    def gather_descs(c, s):
        sl = pl.ds(c * CHUNK, CHUNK)
        return [
            pltpu.make_async_copy(tabs[k].at[idx_v[k].at[sl]],
                                  bufs[s][k], gsems[s])
            for k in range(6)
        ]

    def issue(c, s):
        for d in gather_descs(c, s):
            d.start()

    def drain(c, s):
        for d in gather_descs(c, s):
            d.wait()

    def compute(c, s, lossv_c):
        phb, ptb, prb, nhb, ntb, nrb = bufs[s]

        def group_body(g, lv):
            row0 = c * CHUNK + g * LANES

            def row_body(i, carry):
                nsv, psv = carry
                r = g * LANES + i
                onehot = lane == i
                p_s = _side_score(phb, ptb, prb, r)
                n_s = _side_score(nhb, ntb, nrb, r)
                nsv = jnp.where(onehot,
                                jnp.full((LANES,), -n_s, jnp.float32), nsv)
                psv = jnp.where(onehot,
                                jnp.full((LANES,), p_s, jnp.float32), psv)
                return nsv, psv

            z = jnp.zeros((LANES,), jnp.float32)
            nsv, psv = lax.fori_loop(0, LANES, row_body, (z, z))
            tkv = take_v[pl.ds(row0, LANES)]
            lv = lv + jnp.maximum(0.0, psv + nsv + MARGIN) * tkv
            ns_buf[pl.ds(row0, LANES)] = nsv
            return lv

        return lax.fori_loop(0, CHUNK // LANES, group_body, lossv_c)

    issue(0, 0)

    def pair_body(p, lossv_c):
        c0 = p * 2
        drain(c0, 0)
        issue(c0 + 1, 1)
        lossv_c = compute(c0, 0, lossv_c)
        drain(c0 + 1, 1)

        @pl.when(p < NCHUNK // 2 - 1)
        def _():
            issue(c0 + 2, 0)

        return compute(c0 + 1, 1, lossv_c)

    lossv = lax.fori_loop(0, NCHUNK // 2, pair_body,
                          jnp.zeros((LANES,), jnp.float32))

    pltpu.sync_copy(ns_buf, nscore_out.at[pl.ds(base, ROWS)])
    loss_buf[...] = lossv
    pltpu.sync_copy(loss_buf, partial_out.at[wid])


_disc = functools.partial(
    pl.kernel,
    mesh=plsc.VectorSubcoreMesh(core_axis_name="c", subcore_axis_name="s"),
    compiler_params=pltpu.CompilerParams(needs_layout_passes=False),
    out_type=[
        jax.ShapeDtypeStruct((B,), jnp.float32),
        jax.ShapeDtypeStruct((NW, LANES), jnp.float32),
    ],
    scratch_types=(
        [pltpu.VMEM((ROWS,), jnp.int32) for _ in range(6)]
        + [pltpu.VMEM((CHUNK, PR), jnp.float32) for _ in range(12)]
        + [pltpu.VMEM((ROWS,), jnp.float32),
           pltpu.VMEM((ROWS,), jnp.float32),
           pltpu.VMEM((LANES,), jnp.float32),
           pltpu.SemaphoreType.DMA,
           pltpu.SemaphoreType.DMA,
           pltpu.SemaphoreType.DMA]
    ),
)(_disc_body)


def _repack2_body(xe_ref, xt_ref, o_ref):
    o_ref[:, 0:D] = xe_ref[...].T
    o_ref[:, D:PR] = xt_ref[...].T


def _make_repack2(n_rows, blk):
    grid = -(-n_rows // blk)
    return pl.pallas_call(
        _repack2_body,
        grid=(grid,),
        in_specs=[pl.BlockSpec((D, blk), lambda i: (0, i)),
                  pl.BlockSpec((D, blk), lambda i: (0, i))],
        out_specs=pl.BlockSpec((blk, PR), lambda i: (i, 0)),
        out_shape=jax.ShapeDtypeStruct((grid * blk, PR), jnp.float32),
    )


_repack_ent = _make_repack2(1000000, 16384)
_repack_rel = _make_repack2(1000, 1000)


def _sum_body(p_ref, o_ref):
    o_ref[0, 0] = jnp.sum(p_ref[...])


_sum_partials = pl.pallas_call(
    _sum_body,
    out_shape=jax.ShapeDtypeStruct((1, 1), jnp.float32),
    out_specs=pl.BlockSpec(memory_space=pltpu.SMEM),
)


def kernel(pos_h, pos_r, pos_t, neg_h, neg_r, neg_t, take,
           ent_emb_w, rel_emb_w, ent_transfer_w, rel_transfer_w):
    phys = [s.astype(jnp.int32)
            for s in (pos_h, pos_t, pos_r, neg_h, neg_t, neg_r)]
    takef = take.astype(jnp.float32)
    entc = _repack_ent(ent_emb_w.T, ent_transfer_w.T)
    relc = _repack_rel(rel_emb_w.T, rel_transfer_w.T)
    nscore, partials = _disc(*phys, takef, entc, relc)
    loss = _sum_partials(partials)[0, 0]
    return (loss, nscore)
